# baseline (speedup 1.0000x reference)
import functools

import jax
import jax.numpy as jnp
from jax import lax
from jax.experimental import pallas as pl
from jax.experimental.pallas import tpu as pltpu
from jax.experimental.pallas import tpu_sc as plsc

N_LEVELS = 8
D_MODEL = 2048

try:
    _info = plsc.get_sparse_core_info()
    _NC, _NS = _info.num_cores, _info.num_subcores
except Exception:
    _NC, _NS = 2, 16
_NW = _NC * _NS


@functools.lru_cache(maxsize=None)
def _build(B: int, D: int, V: int):
    b_per_w = B // _NW
    CH = 16
    NBUF = 2
    n_chunks = b_per_w // CH
    n_groups = n_chunks // NBUF
    mesh = plsc.VectorSubcoreMesh(core_axis_name="c", subcore_axis_name="s")

    @functools.partial(
        pl.kernel,
        mesh=mesh,
        compiler_params=pltpu.CompilerParams(needs_layout_passes=False),
        out_type=jax.ShapeDtypeStruct((B, 1, D), jnp.float32),
        scratch_types=[
            pltpu.VMEM((b_per_w,), jnp.int32),
            pltpu.VMEM((V * D,), jnp.float32),
            [pltpu.VMEM((CH, D), jnp.float32) for _ in range(NBUF)],
            [pltpu.SemaphoreType.DMA for _ in range(NBUF)],
        ],
    )
    def lookup_kernel(idx_hbm, rep_hbm, out_hbm, idx_v, table_v, bufs, wsems):
        wid = lax.axis_index("s") * _NC + lax.axis_index("c")
        base = wid * b_per_w
        pltpu.sync_copy(idx_hbm.at[pl.ds(base, b_per_w)], idx_v)
        pltpu.sync_copy(rep_hbm.at[pl.ds(wid * V * D, V * D)], table_v)
        lanes = lax.iota(jnp.int32, 16)

        def group_body(g, carry):
            for b in range(NBUF):
                chunk = g * NBUF + b
                @pl.when(g > 0)
                def _():
                    pltpu.make_async_copy(
                        bufs[b], out_hbm.at[pl.ds(base, CH), 0], wsems[b]
                    ).wait()
                rows = idx_v[pl.ds(chunk * CH, CH)]

                def row_body(i, c2, b=b, rows=rows):
                    rsplat = rows.at[jnp.full((16,), 0, jnp.int32) + i].get(
                        mode="promise_in_bounds")
                    rl = rsplat * D + lanes
                    for k in range(D // 16):
                        val = plsc.load_gather(table_v, [rl + (k * 16)])
                        bufs[b][i, pl.ds(k * 16, 16)] = val
                    return c2

                lax.fori_loop(0, CH, row_body, 0, unroll=False)
                pltpu.async_copy(
                    bufs[b],
                    out_hbm.at[pl.ds(base + chunk * CH, CH), 0],
                    wsems[b])
            return carry

        lax.fori_loop(0, n_groups, group_body, 0, unroll=False)
        for b in range(NBUF):
            pltpu.make_async_copy(
                bufs[b], out_hbm.at[pl.ds(base, CH), 0], wsems[b]).wait()

    return lookup_kernel


def kernel(level_idx, prefix_emb):
    B = level_idx.shape[0]
    V, D = prefix_emb.shape
    rep = jnp.broadcast_to(prefix_emb[None], (_NW, V, D)).reshape(_NW * V * D)
    return _build(B, D, V)(level_idx, rep)


# speedup vs baseline: 1.5515x; 1.5515x over previous
import functools

import jax
import jax.numpy as jnp
from jax import lax
from jax.experimental import pallas as pl
from jax.experimental.pallas import tpu as pltpu
from jax.experimental.pallas import tpu_sc as plsc

N_LEVELS = 8
D_MODEL = 2048

try:
    _info = plsc.get_sparse_core_info()
    _NC, _NS = _info.num_cores, _info.num_subcores
except Exception:
    _NC, _NS = 2, 16
_NW = _NC * _NS


@functools.lru_cache(maxsize=None)
def _build(B: int, D: int, V: int):
    b_per_w = B // _NW
    CH = 16
    NBUF = 2
    n_chunks = b_per_w // CH
    n_groups = n_chunks // NBUF
    mesh = plsc.VectorSubcoreMesh(core_axis_name="c", subcore_axis_name="s")

    @functools.partial(
        pl.kernel,
        mesh=mesh,
        compiler_params=pltpu.CompilerParams(needs_layout_passes=False),
        out_type=jax.ShapeDtypeStruct((B, 1, D), jnp.float32),
        scratch_types=[
            pltpu.VMEM((b_per_w,), jnp.int32),
            pltpu.VMEM((V * D,), jnp.float32),
            [pltpu.VMEM((CH, D), jnp.float32) for _ in range(NBUF)],
            [pltpu.SemaphoreType.DMA for _ in range(NBUF)],
        ],
    )
    def lookup_kernel(idx_hbm, rep_hbm, out_hbm, idx_v, table_v, bufs, wsems):
        wid = lax.axis_index("s") * _NC + lax.axis_index("c")
        base = wid * b_per_w
        pltpu.sync_copy(idx_hbm.at[pl.ds(base, b_per_w)], idx_v)
        pltpu.sync_copy(rep_hbm.at[pl.ds(wid * V * D, V * D)], table_v)
        lanes = lax.iota(jnp.int32, 16)

        def group_body(g, carry):
            for b in range(NBUF):
                chunk = g * NBUF + b
                @pl.when(g > 0)
                def _():
                    pltpu.make_async_copy(
                        bufs[b], out_hbm.at[pl.ds(base, CH), 0], wsems[b]
                    ).wait()
                rows = idx_v[pl.ds(chunk * CH, CH)]

                @plsc.parallel_loop(0, CH, 1, unroll=2)
                def _(i, b=b, rows=rows):
                    rsplat = rows.at[jnp.full((16,), 0, jnp.int32) + i].get(
                        mode="promise_in_bounds")
                    rl = rsplat * D + lanes
                    for k in range(D // 16):
                        val = plsc.load_gather(table_v, [rl + (k * 16)])
                        bufs[b][i, pl.ds(k * 16, 16)] = val
                pltpu.async_copy(
                    bufs[b],
                    out_hbm.at[pl.ds(base + chunk * CH, CH), 0],
                    wsems[b])
            return carry

        lax.fori_loop(0, n_groups, group_body, 0, unroll=False)
        for b in range(NBUF):
            pltpu.make_async_copy(
                bufs[b], out_hbm.at[pl.ds(base, CH), 0], wsems[b]).wait()

    return lookup_kernel


def kernel(level_idx, prefix_emb):
    B = level_idx.shape[0]
    V, D = prefix_emb.shape
    rep = jnp.broadcast_to(prefix_emb[None], (_NW, V, D)).reshape(_NW * V * D)
    return _build(B, D, V)(level_idx, rep)
